# baseline (speedup 1.0000x reference)
import functools

import jax
import jax.numpy as jnp
from jax.experimental import pallas as pl

B, S, W = 4, 2048, 1024
LN_EPS = 1e-3
S_BLK = 128


def _fused_kernel(x_ref, idf_ref, tt_ref, pos_ref, g_ref, b_ref, o_ref):
    x = x_ref[...]
    idf = idf_ref[...]
    row0 = tt_ref[0, :]
    row1 = tt_ref[1, :]
    pos = pos_ref[...]
    y = x + pos[None, :, :] + row0[None, None, :]
    y = y + idf[:, :, None] * (row1 - row0)[None, None, :]
    mean = jnp.mean(y, axis=-1, keepdims=True)
    yc = y - mean
    var = jnp.mean(yc * yc, axis=-1, keepdims=True)
    out = yc * jax.lax.rsqrt(var + LN_EPS)
    out = out * g_ref[...][None, None, :] + b_ref[...][None, None, :]
    o_ref[...] = out


@functools.partial(jax.jit, static_argnames=())
def _run(x, idf, tt, pos, gamma, beta):
    grid = (S // S_BLK,)
    return pl.pallas_call(
        _fused_kernel,
        grid=grid,
        in_specs=[
            pl.BlockSpec((B, S_BLK, W), lambda i: (0, i, 0)),
            pl.BlockSpec((B, S_BLK), lambda i: (0, i)),
            pl.BlockSpec((2, W), lambda i: (0, 0)),
            pl.BlockSpec((S_BLK, W), lambda i: (i, 0)),
            pl.BlockSpec((W,), lambda i: (0,)),
            pl.BlockSpec((W,), lambda i: (0,)),
        ],
        out_specs=pl.BlockSpec((B, S_BLK, W), lambda i: (0, i, 0)),
        out_shape=jax.ShapeDtypeStruct((B, S_BLK * grid[0], W), jnp.float32),
    )(x, idf, tt, pos, gamma, beta)


def kernel(input_tensor, token_type_ids, token_type_table, full_position_embeddings, ln_gamma, ln_beta):
    idf = token_type_ids.astype(jnp.float32)
    pos = full_position_embeddings[:S, :]
    return _run(input_tensor, idf, token_type_table, pos, ln_gamma, ln_beta)


# speedup vs baseline: 1.1087x; 1.1087x over previous
import functools

import jax
import jax.numpy as jnp
from jax.experimental import pallas as pl

B, S, W = 4, 2048, 1024
LN_EPS = 1e-3
S_BLK = 512


def _fused_kernel(x_ref, idf_ref, tt_ref, pos_ref, g_ref, b_ref, o_ref):
    x = x_ref[...]
    idf = idf_ref[...]
    row0 = tt_ref[0, :]
    row1 = tt_ref[1, :]
    pos = pos_ref[...]
    y = x + pos[None, :, :] + row0[None, None, :]
    y = y + idf[:, :, None] * (row1 - row0)[None, None, :]
    mean = jnp.mean(y, axis=-1, keepdims=True)
    yc = y - mean
    var = jnp.mean(yc * yc, axis=-1, keepdims=True)
    out = yc * jax.lax.rsqrt(var + LN_EPS)
    out = out * g_ref[...][None, None, :] + b_ref[...][None, None, :]
    o_ref[...] = out


@functools.partial(jax.jit, static_argnames=())
def _run(x, idf, tt, pos, gamma, beta):
    grid = (S // S_BLK,)
    return pl.pallas_call(
        _fused_kernel,
        grid=grid,
        in_specs=[
            pl.BlockSpec((B, S_BLK, W), lambda i: (0, i, 0)),
            pl.BlockSpec((B, S_BLK), lambda i: (0, i)),
            pl.BlockSpec((2, W), lambda i: (0, 0)),
            pl.BlockSpec((S_BLK, W), lambda i: (i, 0)),
            pl.BlockSpec((W,), lambda i: (0,)),
            pl.BlockSpec((W,), lambda i: (0,)),
        ],
        out_specs=pl.BlockSpec((B, S_BLK, W), lambda i: (0, i, 0)),
        out_shape=jax.ShapeDtypeStruct((B, S_BLK * grid[0], W), jnp.float32),
    )(x, idf, tt, pos, gamma, beta)


def kernel(input_tensor, token_type_ids, token_type_table, full_position_embeddings, ln_gamma, ln_beta):
    idf = token_type_ids.astype(jnp.float32)
    pos = full_position_embeddings[:S, :]
    return _run(input_tensor, idf, token_type_table, pos, ln_gamma, ln_beta)
